# baseline (speedup 1.0000x reference)
import functools

import jax
import jax.numpy as jnp
from jax import lax
from jax.experimental import pallas as pl
from jax.experimental.pallas import tpu as pltpu
from jax.experimental.pallas import tpu_sc as plsc

_NC = 2
_NS = 16
_L = 16


def _make_sc_kernel(S, B, D, CS):
    n_workers = _NC * _NS
    rows_per_w = S // n_workers
    n_chunks = rows_per_w // CS
    mesh = plsc.VectorSubcoreMesh(
        core_axis_name="c", subcore_axis_name="s",
        num_cores=_NC, num_subcores=_NS,
    )

    @functools.partial(
        pl.kernel,
        out_type=jax.ShapeDtypeStruct((S, B, D), jnp.float32),
        mesh=mesh,
        scratch_types=[
            pltpu.VMEM((CS, B, D), jnp.float32),
            pltpu.VMEM((CS, D), jnp.float32),
            pltpu.SemaphoreType.DMA,
            pltpu.SemaphoreType.DMA,
        ],
    )
    def sc_kernel(x_hbm, pos_hbm, out_hbm, xbuf, pbuf, semx, semp):
        wid = lax.axis_index("s") * _NC + lax.axis_index("c")
        base = wid * rows_per_w

        @pl.loop(0, n_chunks)
        def _chunk(i):
            s0 = base + i * CS
            cx = pltpu.async_copy(x_hbm.at[pl.ds(s0, CS)], xbuf, semx)
            cp = pltpu.async_copy(pos_hbm.at[pl.ds(s0, CS)], pbuf, semp)
            cp.wait()
            cx.wait()

            @pl.loop(0, D // _L)
            def _vec(k):
                sl = pl.ds(k * _L, _L)
                for r in range(CS):
                    p = pbuf[r, sl]
                    for b in range(B):
                        xbuf[r, b, sl] = xbuf[r, b, sl] + p

            pltpu.sync_copy(xbuf, out_hbm.at[pl.ds(s0, CS)])

    return sc_kernel


def kernel(x, pos_table):
    S, B, D = x.shape
    return _make_sc_kernel(S, B, D, CS=4)(x, pos_table)


# speedup vs baseline: 1.4063x; 1.4063x over previous
import functools

import jax
import jax.numpy as jnp
from jax import lax
from jax.experimental import pallas as pl
from jax.experimental.pallas import tpu as pltpu
from jax.experimental.pallas import tpu_sc as plsc

_NC = 2
_NS = 16
_L = 16


def _make_sc_kernel(S, B, D, CS):
    n_workers = _NC * _NS
    rows_per_w = S // n_workers
    n_chunks = rows_per_w // CS
    mesh = plsc.VectorSubcoreMesh(
        core_axis_name="c", subcore_axis_name="s",
        num_cores=_NC, num_subcores=_NS,
    )

    @functools.partial(
        pl.kernel,
        out_type=jax.ShapeDtypeStruct((S, B, D), jnp.float32),
        mesh=mesh,
        scratch_types=[
            pltpu.VMEM((CS, B, D), jnp.float32),
            pltpu.VMEM((CS, B, D), jnp.float32),
            pltpu.VMEM((CS, D), jnp.float32),
            pltpu.VMEM((CS, D), jnp.float32),
            pltpu.SemaphoreType.DMA,
            pltpu.SemaphoreType.DMA,
            pltpu.SemaphoreType.DMA,
            pltpu.SemaphoreType.DMA,
            pltpu.SemaphoreType.DMA,
            pltpu.SemaphoreType.DMA,
        ],
    )
    def sc_kernel(x_hbm, pos_hbm, out_hbm,
                  xb0, xb1, pb0, pb1,
                  slx0, slx1, slp0, slp1, sst0, sst1):
        xb = (xb0, xb1)
        pb = (pb0, pb1)
        slx = (slx0, slx1)
        slp = (slp0, slp1)
        sst = (sst0, sst1)

        wid = lax.axis_index("s") * _NC + lax.axis_index("c")
        base = wid * rows_per_w

        def start_load(j, b):
            s0 = base + j * CS
            pltpu.async_copy(x_hbm.at[pl.ds(s0, CS)], xb[b], slx[b])
            pltpu.async_copy(pos_hbm.at[pl.ds(s0, CS)], pb[b], slp[b])

        def wait_load(b):
            pltpu.make_async_copy(x_hbm.at[pl.ds(0, CS)], xb[b], slx[b]).wait()
            pltpu.make_async_copy(pos_hbm.at[pl.ds(0, CS)], pb[b], slp[b]).wait()

        def start_store(j, b):
            s0 = base + j * CS
            pltpu.async_copy(xb[b], out_hbm.at[pl.ds(s0, CS)], sst[b])

        def wait_store(b):
            pltpu.make_async_copy(xb[b], out_hbm.at[pl.ds(0, CS)], sst[b]).wait()

        def compute(b):
            @pl.loop(0, D // _L)
            def _vec(k):
                sl = pl.ds(k * _L, _L)
                for r in range(CS):
                    p = pb[b][r, sl]
                    for bb in range(B):
                        xb[b][r, bb, sl] = xb[b][r, bb, sl] + p

        start_load(0, 0)

        @pl.loop(0, n_chunks, step=2)
        def _chunk(i):
            for b in (0, 1):
                j = i + b
                nb = 1 - b

                @pl.when(j >= 1)
                def _ws():
                    wait_store(nb)

                @pl.when(j + 1 < n_chunks)
                def _ld():
                    start_load(j + 1, nb)

                wait_load(b)
                compute(b)
                start_store(j, b)

        wait_store((n_chunks - 1) % 2)

    return sc_kernel


def kernel(x, pos_table):
    S, B, D = x.shape
    return _make_sc_kernel(S, B, D, CS=4)(x, pos_table)
